# baseline (speedup 1.0000x reference)
import functools

import jax
import jax.numpy as jnp
from jax import lax
from jax.experimental import pallas as pl
from jax.experimental.pallas import tpu as pltpu
from jax.experimental.pallas import tpu_sc as plsc

CH = 128


def _degree_factors(d0, d1):
    cnt = d0 + d1 + 1.0
    inv_deg = 1.0 / (cnt + 1e-8)
    deg2 = cnt * inv_deg
    g = lax.rsqrt(deg2 + 1e-8)
    return inv_deg, g


def _make_sc_kernels(n, d, e, nc, ns):
    nw = nc * ns
    rpt = n // ns
    assert e % CH == 0 and n % (ns * 8) == 0
    nrows = e // CH
    base = nrows // nw
    assert nrows % nw == 0 and base % 8 == 0 and base % 4 == 0

    mesh = plsc.VectorSubcoreMesh(core_axis_name="c", subcore_axis_name="s")

    @functools.partial(
        pl.kernel,
        out_type=jax.ShapeDtypeStruct((2 * n,), jnp.float32),
        mesh=mesh,
        scratch_types=[
            pltpu.VMEM_SHARED((n,), jnp.float32),
            pltpu.VMEM((CH,), jnp.float32),
            pltpu.VMEM((base, CH), jnp.int32),
            pltpu.VMEM((rpt,), jnp.float32),
            pltpu.SemaphoreType.DMA,
        ],
    )
    def deg_kernel(row2_hbm, z1_hbm, ones1_hbm, deg_hbm,
                   deg_sh, onesv, rowv_all, zbuf, sem):
        c = lax.axis_index("c")
        s = lax.axis_index("s")
        wid = c * ns + s
        row0 = s * rpt
        pltpu.sync_copy(z1_hbm, zbuf)
        pltpu.sync_copy(zbuf, deg_sh.at[pl.ds(row0, rpt)])
        pltpu.sync_copy(ones1_hbm, onesv)
        rbase = wid * base
        pltpu.sync_copy(row2_hbm.at[pl.ds(rbase, base)], rowv_all)
        plsc.subcore_barrier()

        def fire(i, carry):
            pltpu.async_copy(onesv, deg_sh.at[rowv_all.at[i]], sem, add=True)
            return carry

        def drain(i, carry):
            pltpu.make_async_copy(onesv, deg_sh.at[rowv_all.at[i]],
                                  sem).wait()
            return carry

        gdepth = 10 if base % 10 == 0 else 2
        assert base % gdepth == 0

        def group(gi, carry):
            lax.fori_loop(gi * gdepth, (gi + 1) * gdepth, fire, 0)
            lax.fori_loop(gi * gdepth, (gi + 1) * gdepth, drain, 0)
            return carry

        lax.fori_loop(0, base // gdepth, group, 0)
        plsc.subcore_barrier()
        pltpu.sync_copy(deg_sh.at[pl.ds(row0, rpt)], zbuf)
        pltpu.sync_copy(zbuf, deg_hbm.at[pl.ds(c * n + row0, rpt)])

    @functools.partial(
        pl.kernel,
        out_type=jax.ShapeDtypeStruct((2, n, d), jnp.float32),
        mesh=mesh,
        scratch_types=[
            pltpu.VMEM_SHARED((n, d), jnp.float32),
            pltpu.VMEM((base, CH), jnp.int32),
            pltpu.VMEM((CH, d), jnp.float32),
            pltpu.VMEM((CH, d), jnp.float32),
            pltpu.VMEM((CH,), jnp.int32),
            pltpu.VMEM((CH,), jnp.int32),
            pltpu.VMEM((CH,), jnp.int32),
            pltpu.VMEM((CH,), jnp.int32),
            pltpu.VMEM((CH,), jnp.int32),
            pltpu.VMEM((CH,), jnp.int32),
            pltpu.VMEM((CH,), jnp.int32),
            pltpu.VMEM((CH,), jnp.int32),
            pltpu.SemaphoreType.DMA,
            pltpu.SemaphoreType.DMA,
            pltpu.SemaphoreType.DMA,
            pltpu.SemaphoreType.DMA,
        ],
    )
    def spmv_kernel(y_hbm, pk2_hbm, zrows_hbm, s_hbm,
                    acc_sh, pk_all, gbuf0, gbuf1,
                    r0, c0, r1, c1, r2, c2, r3, c3,
                    semg0, semg1, sems0, sems1):
        rowp = (r0, r1, r2, r3)
        colp = (c0, c1, c2, c3)
        c = lax.axis_index("c")
        s = lax.axis_index("s")
        wid = c * ns + s
        row0 = s * rpt
        pltpu.sync_copy(zrows_hbm, acc_sh.at[pl.ds(row0, rpt)])
        rbase = wid * base
        pltpu.sync_copy(pk2_hbm.at[pl.ds(rbase, base)], pk_all)
        plsc.subcore_barrier()

        def unpack(i, m):
            def lane(j, carry):
                v = pk_all[i, pl.ds(j * 16, 16)]
                rowp[m][pl.ds(j * 16, 16)] = jnp.bitwise_and(v, 0xFFFF)
                colp[m][pl.ds(j * 16, 16)] = jnp.right_shift(v, 16)
                return carry

            lax.fori_loop(0, CH // 16, lane, 0)

        gbuf = (gbuf0, gbuf1)
        semg = (semg0, semg1)
        sems = (sems0, sems1)

        def start_g(m, b):
            pltpu.async_copy(y_hbm.at[colp[m]], gbuf[b], semg[b])

        def wait_g(m, b):
            pltpu.make_async_copy(y_hbm.at[colp[m]], gbuf[b],
                                  semg[b]).wait()

        def start_s(m, b):
            pltpu.async_copy(gbuf[b], acc_sh.at[rowp[m]], sems[b], add=True)

        def wait_s(m, b):
            pltpu.make_async_copy(gbuf[b], acc_sh.at[rowp[m]],
                                  sems[b]).wait()

        nouter = base // 4
        unpack(0, 0)
        unpack(1, 1)
        start_g(0, 0)

        def body(g, carry):
            last = g + 1 >= nouter
            for m in range(4):
                j = 4 * g + m
                b = m % 2
                wait_g(m, b)
                start_s(m, b)
                if m < 2:
                    unpack(j + 2, (m + 2) % 4)
                else:
                    @pl.when(jnp.logical_not(last))
                    def _():
                        unpack(j + 2, (m + 2) % 4)

                if m == 0:
                    @pl.when(g > 0)
                    def _():
                        wait_s(3, 1 - b)
                else:
                    wait_s(m - 1, 1 - b)
                if m < 3:
                    start_g(m + 1, 1 - b)
                else:
                    @pl.when(jnp.logical_not(last))
                    def _():
                        start_g(0, 1 - b)
            return carry

        lax.fori_loop(0, nouter, body, 0)
        wait_s(3, 1)
        plsc.subcore_barrier()
        pltpu.sync_copy(acc_sh.at[pl.ds(row0, rpt)],
                        s_hbm.at[c, pl.ds(row0, rpt)])

    return deg_kernel, spmv_kernel


def _make_tc_kernels(n, d, br):
    grid = (n // br,)
    row_spec = pl.BlockSpec((br, d), lambda i: (i, 0))
    deg_spec = pl.BlockSpec((br, 1), lambda i: (i, 0))
    s_spec = pl.BlockSpec((2, br, d), lambda i: (0, i, 0))
    fdt = jax.ShapeDtypeStruct((n, d), jnp.float32)

    def scale0_body(x_ref, d0_ref, d1_ref, y_ref):
        _, g = _degree_factors(d0_ref[...], d1_ref[...])
        y_ref[...] = g * x_ref[...]

    scale0 = pl.pallas_call(
        scale0_body, grid=grid,
        in_specs=[row_spec, deg_spec, deg_spec],
        out_specs=row_spec, out_shape=fdt)

    def update_body(s_ref, y_ref, d0_ref, d1_ref, y2_ref):
        inv_deg, g = _degree_factors(d0_ref[...], d1_ref[...])
        p = inv_deg * g * g
        y2_ref[...] = p * (s_ref[0] + s_ref[1] + y_ref[...])

    update = pl.pallas_call(
        update_body, grid=grid,
        in_specs=[s_spec, row_spec, deg_spec, deg_spec],
        out_specs=row_spec, out_shape=fdt)

    def final_body(s_ref, y0_ref, y1_ref, y2_ref, d0_ref, d1_ref, o_ref):
        inv_deg, g = _degree_factors(d0_ref[...], d1_ref[...])
        p = inv_deg * g * g
        y3 = p * (s_ref[0] + s_ref[1] + y2_ref[...])
        o_ref[...] = (y0_ref[...] + y1_ref[...] + y2_ref[...] + y3) \
            * (0.25 / g)

    final = pl.pallas_call(
        final_body, grid=grid,
        in_specs=[s_spec, row_spec, row_spec, row_spec, deg_spec, deg_spec],
        out_specs=row_spec, out_shape=fdt)

    return scale0, update, final


def kernel(x, edge_index):
    n, d = x.shape
    e = edge_index.shape[1]
    try:
        info = plsc.get_sparse_core_info()
        nc, ns = info.num_cores, info.num_subcores
    except Exception:
        nc, ns = 2, 16
    nw = nc * ns
    n_pad = -(-n // (ns * 8)) * (ns * 8)
    e_pad = -(-e // (nw * 8 * CH)) * (nw * 8 * CH)
    if e_pad != e and n_pad == n:
        n_pad += ns * 8
    rpt = n_pad // ns

    deg_k, spmv_k = _make_sc_kernels(n_pad, d, e_pad, nc, ns)
    scale0, update, final = _make_tc_kernels(n_pad, d, br=n_pad // 8)

    assert n_pad < 2 ** 15
    row = edge_index[0]
    col = edge_index[1]
    if e_pad != e:
        dummy = n + (jnp.arange(e_pad - e, dtype=jnp.int32) % (n_pad - n))
        row = jnp.concatenate([row, dummy])
        col = jnp.concatenate([col, dummy])
    row2 = row.reshape(e_pad // CH, CH)
    pk2 = (row | (col << 16)).reshape(e_pad // CH, CH)
    xp = jnp.pad(x, ((0, n_pad - n), (0, 0))) if n_pad != n else x
    zrows = jnp.zeros((rpt, d), jnp.float32)
    z1 = jnp.zeros((rpt,), jnp.float32)
    ones1 = jnp.ones((CH,), jnp.float32)

    deg = deg_k(row2, z1, ones1)
    d0 = deg[:n_pad, None]
    d1 = deg[n_pad:, None]
    ys = [scale0(xp, d0, d1)]
    for k in range(2):
        s = spmv_k(ys[-1], pk2, zrows)
        ys.append(update(s, ys[-1], d0, d1))
    s = spmv_k(ys[-1], pk2, zrows)
    out = final(s, ys[0], ys[1], ys[2], d0, d1)
    return out[:n] if n_pad != n else out


# speedup vs baseline: 1.1695x; 1.1695x over previous
import functools

import jax
import jax.numpy as jnp
from jax import lax
from jax.experimental import pallas as pl
from jax.experimental.pallas import tpu as pltpu
from jax.experimental.pallas import tpu_sc as plsc

CH = 128


def _degree_factors(d0, d1):
    cnt = d0 + d1 + 1.0
    inv_deg = 1.0 / (cnt + 1e-8)
    deg2 = cnt * inv_deg
    g = lax.rsqrt(deg2 + 1e-8)
    return inv_deg, g


def _make_sc_kernels(n, d, e, nc, ns):
    nw = nc * ns
    rpt = n // ns
    assert e % CH == 0 and n % (ns * 8) == 0
    nrows = e // CH
    base = nrows // nw
    assert nrows % nw == 0 and base % 2 == 0 and base % 8 == 0

    mesh = plsc.VectorSubcoreMesh(core_axis_name="c", subcore_axis_name="s")

    @functools.partial(
        pl.kernel,
        out_type=jax.ShapeDtypeStruct((2 * n,), jnp.float32),
        mesh=mesh,
        scratch_types=[
            pltpu.VMEM_SHARED((n,), jnp.float32),
            pltpu.VMEM((CH,), jnp.float32),
            pltpu.VMEM((base, CH), jnp.int32),
            pltpu.VMEM((rpt,), jnp.float32),
            pltpu.SemaphoreType.DMA,
        ],
    )
    def deg_kernel(row2_hbm, z1_hbm, ones1_hbm, deg_hbm,
                   deg_sh, onesv, rowv_all, zbuf, sem):
        c = lax.axis_index("c")
        s = lax.axis_index("s")
        wid = c * ns + s
        row0 = s * rpt
        pltpu.sync_copy(z1_hbm, zbuf)
        pltpu.sync_copy(zbuf, deg_sh.at[pl.ds(row0, rpt)])
        pltpu.sync_copy(ones1_hbm, onesv)
        rbase = wid * base
        pltpu.sync_copy(row2_hbm.at[pl.ds(rbase, base)], rowv_all)
        plsc.subcore_barrier()

        def fire(i, carry):
            pltpu.async_copy(onesv, deg_sh.at[rowv_all.at[i]], sem, add=True)
            return carry

        def drain(i, carry):
            pltpu.make_async_copy(onesv, deg_sh.at[rowv_all.at[i]],
                                  sem).wait()
            return carry

        gdepth = 10 if base % 10 == 0 else 2
        assert base % gdepth == 0

        def group(gi, carry):
            lax.fori_loop(gi * gdepth, (gi + 1) * gdepth, fire, 0)
            lax.fori_loop(gi * gdepth, (gi + 1) * gdepth, drain, 0)
            return carry

        lax.fori_loop(0, base // gdepth, group, 0)
        plsc.subcore_barrier()
        pltpu.sync_copy(deg_sh.at[pl.ds(row0, rpt)], zbuf)
        pltpu.sync_copy(zbuf, deg_hbm.at[pl.ds(c * n + row0, rpt)])

    @functools.partial(
        pl.kernel,
        out_type=jax.ShapeDtypeStruct((2, n, d), jnp.float32),
        mesh=mesh,
        scratch_types=[
            pltpu.VMEM_SHARED((n, d), jnp.float32),
            pltpu.VMEM((base, CH), jnp.int32),
            pltpu.VMEM((CH, d), jnp.float32),
            pltpu.VMEM((CH, d), jnp.float32),
            pltpu.VMEM((CH,), jnp.int32),
            pltpu.VMEM((CH,), jnp.int32),
            pltpu.VMEM((CH,), jnp.int32),
            pltpu.VMEM((CH,), jnp.int32),
            pltpu.SemaphoreType.DMA,
            pltpu.SemaphoreType.DMA,
        ],
    )
    def spmv_kernel(y_hbm, pk2_hbm, zrows_hbm, s_hbm,
                    acc_sh, pk_all, gbuf0, gbuf1,
                    rowv0, colv0, rowv1, colv1, sem0, sem1):
        c = lax.axis_index("c")
        s = lax.axis_index("s")
        wid = c * ns + s
        row0 = s * rpt
        pltpu.sync_copy(zrows_hbm, acc_sh.at[pl.ds(row0, rpt)])
        rbase = wid * base
        pltpu.sync_copy(pk2_hbm.at[pl.ds(rbase, base)], pk_all)
        plsc.subcore_barrier()

        def unpack(i, rowv, colv):
            def lane(j, carry):
                v = pk_all[i, pl.ds(j * 16, 16)]
                rowv[pl.ds(j * 16, 16)] = jnp.bitwise_and(v, 0xFFFF)
                colv[pl.ds(j * 16, 16)] = jnp.right_shift(v, 16)
                return carry

            lax.fori_loop(0, CH // 16, lane, 0)

        def start_g(buf, colv, sem):
            pltpu.async_copy(y_hbm.at[colv], buf, sem)

        def wait_g(buf, colv, sem):
            pltpu.make_async_copy(y_hbm.at[colv], buf, sem).wait()

        def scat(buf, rowv):
            pltpu.sync_copy(buf, acc_sh.at[rowv], add=True)

        nh = base // 2
        unpack(0, rowv0, colv0)
        start_g(gbuf0, colv0, sem0)

        def body(g, carry):
            i0 = 2 * g
            unpack(i0 + 1, rowv1, colv1)
            start_g(gbuf1, colv1, sem1)
            wait_g(gbuf0, colv0, sem0)
            scat(gbuf0, rowv0)

            @pl.when(g + 1 < nh)
            def _():
                unpack(i0 + 2, rowv0, colv0)
                start_g(gbuf0, colv0, sem0)

            wait_g(gbuf1, colv1, sem1)
            scat(gbuf1, rowv1)
            return carry

        lax.fori_loop(0, nh, body, 0)
        plsc.subcore_barrier()
        pltpu.sync_copy(acc_sh.at[pl.ds(row0, rpt)],
                        s_hbm.at[c, pl.ds(row0, rpt)])

    return deg_kernel, spmv_kernel


def _make_tc_kernels(n, d, br):
    grid = (n // br,)
    row_spec = pl.BlockSpec((br, d), lambda i: (i, 0))
    deg_spec = pl.BlockSpec((br, 1), lambda i: (i, 0))
    s_spec = pl.BlockSpec((2, br, d), lambda i: (0, i, 0))
    fdt = jax.ShapeDtypeStruct((n, d), jnp.float32)

    def scale0_body(x_ref, d0_ref, d1_ref, y_ref):
        _, g = _degree_factors(d0_ref[...], d1_ref[...])
        y_ref[...] = g * x_ref[...]

    scale0 = pl.pallas_call(
        scale0_body, grid=grid,
        in_specs=[row_spec, deg_spec, deg_spec],
        out_specs=row_spec, out_shape=fdt)

    def update_body(s_ref, y_ref, d0_ref, d1_ref, y2_ref):
        inv_deg, g = _degree_factors(d0_ref[...], d1_ref[...])
        p = inv_deg * g * g
        y2_ref[...] = p * (s_ref[0] + s_ref[1] + y_ref[...])

    update = pl.pallas_call(
        update_body, grid=grid,
        in_specs=[s_spec, row_spec, deg_spec, deg_spec],
        out_specs=row_spec, out_shape=fdt)

    def final_body(s_ref, y0_ref, y1_ref, y2_ref, d0_ref, d1_ref, o_ref):
        inv_deg, g = _degree_factors(d0_ref[...], d1_ref[...])
        p = inv_deg * g * g
        y3 = p * (s_ref[0] + s_ref[1] + y2_ref[...])
        o_ref[...] = (y0_ref[...] + y1_ref[...] + y2_ref[...] + y3) \
            * (0.25 / g)

    final = pl.pallas_call(
        final_body, grid=grid,
        in_specs=[s_spec, row_spec, row_spec, row_spec, deg_spec, deg_spec],
        out_specs=row_spec, out_shape=fdt)

    return scale0, update, final


def kernel(x, edge_index):
    n, d = x.shape
    e = edge_index.shape[1]
    try:
        info = plsc.get_sparse_core_info()
        nc, ns = info.num_cores, info.num_subcores
    except Exception:
        nc, ns = 2, 16
    nw = nc * ns
    n_pad = -(-n // (ns * 8)) * (ns * 8)
    e_pad = -(-e // (nw * 8 * CH)) * (nw * 8 * CH)
    if e_pad != e and n_pad == n:
        n_pad += ns * 8
    rpt = n_pad // ns

    deg_k, spmv_k = _make_sc_kernels(n_pad, d, e_pad, nc, ns)
    scale0, update, final = _make_tc_kernels(n_pad, d, br=n_pad // 8)

    assert n_pad < 2 ** 15
    row = edge_index[0]
    col = edge_index[1]
    if e_pad != e:
        dummy = n + (jnp.arange(e_pad - e, dtype=jnp.int32) % (n_pad - n))
        row = jnp.concatenate([row, dummy])
        col = jnp.concatenate([col, dummy])
    row2 = row.reshape(e_pad // CH, CH)
    pk2 = (row | (col << 16)).reshape(e_pad // CH, CH)
    xp = jnp.pad(x, ((0, n_pad - n), (0, 0))) if n_pad != n else x
    zrows = jnp.zeros((rpt, d), jnp.float32)
    z1 = jnp.zeros((rpt,), jnp.float32)
    ones1 = jnp.ones((CH,), jnp.float32)

    deg = deg_k(row2, z1, ones1)
    d0 = deg[:n_pad, None]
    d1 = deg[n_pad:, None]
    ys = [scale0(xp, d0, d1)]
    for k in range(2):
        s = spmv_k(ys[-1], pk2, zrows)
        ys.append(update(s, ys[-1], d0, d1))
    s = spmv_k(ys[-1], pk2, zrows)
    out = final(s, ys[0], ys[1], ys[2], d0, d1)
    return out[:n] if n_pad != n else out
